# baseline (speedup 1.0000x reference)
import jax
import jax.numpy as jnp
from jax import lax
from jax.experimental import pallas as pl
from jax.experimental.pallas import tpu as pltpu
from jax.experimental.pallas import tpu_sc as plsc

N = 100000
D = 128
G = 256

NC = 2
NS = 16
L = 16
DH = D // NC
CH = 128
NBUF = 6
PF = 3
UNROLL = 8

NFULL = N // CH
TAIL = N - NFULL * CH
TAIL_OFF = NFULL * CH
CPT = 49
LPT = NFULL - (NS - 1) * CPT
AW = DH + L
ZR = G // NS
GPT = G // NS


def _seg_mean_kernel(x_hbm, b2d_hbm, btail_hbm, out_hbm,
                     acc_sh, acc_v, rows_v, ids_v, idx_v, cnt_v,
                     sem_i, sem0, sem1, sem2, sem3, sem4, sem5, sem_s):
  tid = lax.axis_index("s")
  cid = lax.axis_index("c")
  c0 = cid * DH
  chunk0 = tid * CPT
  n_chunks = jnp.where(tid < NS - 1, CPT, LPT)
  sems = (sem0, sem1, sem2, sem3, sem4, sem5)
  zero16 = jnp.zeros((L,), jnp.float32)
  one16 = jnp.ones((L,), jnp.float32)

  @pl.when(tid < NS - 1)
  def _ids_full():
    pltpu.async_copy(b2d_hbm.at[pl.ds(chunk0, CPT)], ids_v.at[pl.ds(0, CPT)],
                     sem_i)

  @pl.when(tid == NS - 1)
  def _ids_last():
    pltpu.async_copy(b2d_hbm.at[pl.ds(chunk0, LPT)], ids_v.at[pl.ds(0, LPT)],
                     sem_i)

  def zero_body(r, _):
    for q in range(AW // L):
      acc_v[r, pl.ds(q * L, L)] = zero16
    return 0
  lax.fori_loop(0, G, zero_body, 0)
  pltpu.sync_copy(acc_v.at[pl.ds(0, ZR)], acc_sh.at[pl.ds(tid * ZR, ZR)])

  base = jnp.arange(L, dtype=jnp.int32)
  for h in range(2):
    for r in range(CH // L):
      idx_v[h, pl.ds(r * L, L)] = base + (h * CH + r * L)

  @pl.when(tid < NS - 1)
  def _ids_full_wait():
    pltpu.make_async_copy(b2d_hbm.at[pl.ds(chunk0, CPT)],
                          ids_v.at[pl.ds(0, CPT)], sem_i).wait()

  @pl.when(tid == NS - 1)
  def _ids_last_wait():
    pltpu.make_async_copy(b2d_hbm.at[pl.ds(chunk0, LPT)],
                          ids_v.at[pl.ds(0, LPT)], sem_i).wait()

  plsc.subcore_barrier()

  def row_copy(k, b):
    off = (chunk0 + k) * CH
    return pltpu.make_async_copy(x_hbm.at[pl.ds(off, CH), pl.ds(c0, DH)],
                                 rows_v.at[b], sems[b])

  for b in range(PF):
    row_copy(b, b).start()

  sixteen16 = jnp.full((L,), float(L), jnp.float32)
  chunk16 = jnp.full((L,), float(CH), jnp.float32)
  zq = (zero16,) * (DH // L)

  def accum_groups(b, kk, nrows):
    def acc_body(ri, _):
      idvec = ids_v[kk, pl.ds(ri * L, L)]
      sid0 = jnp.max(idvec)
      uniform = sid0 == jnp.min(idvec)

      @pl.when(uniform)
      def _fast():
        for q in range(DH // L):
          s = rows_v[b, ri * L, pl.ds(q * L, L)]
          for u in range(1, L):
            s = s + rows_v[b, ri * L + u, pl.ds(q * L, L)]
          plsc.addupdate(acc_v.at[sid0, pl.ds(q * L, L)], s)
        plsc.addupdate(acc_v.at[sid0, pl.ds(DH, L)], sixteen16)

      @pl.when(jnp.logical_not(uniform))
      def _slow():
        for u in range(L):
          r = ri * L + u
          sid = idvec[u]
          for q in range(DH // L):
            plsc.addupdate(acc_v.at[sid, pl.ds(q * L, L)],
                           rows_v[b, r, pl.ds(q * L, L)])
          plsc.addupdate(acc_v.at[sid, pl.ds(DH, L)], one16)
      return 0
    lax.fori_loop(0, nrows // L, acc_body, 0)

  def accum_chunk(b, kk):
    first = jnp.min(ids_v[kk, pl.ds(0, L)])
    last = jnp.max(ids_v[kk, pl.ds(CH - L, L)])

    @pl.when(first == last)
    def _chunk_fast():
      sums = list(zq)
      for r in range(CH):
        for q in range(DH // L):
          sums[q] = sums[q] + rows_v[b, r, pl.ds(q * L, L)]
      for q in range(DH // L):
        plsc.addupdate(acc_v.at[last, pl.ds(q * L, L)], sums[q])
      plsc.addupdate(acc_v.at[last, pl.ds(DH, L)], chunk16)

    @pl.when(first != last)
    def _chunk_slow():
      accum_groups(b, kk, CH)

  def ring_body(i, _):
    for b in range(NBUF):
      k = NBUF * i + b

      @pl.when(k < n_chunks)
      def _process():
        row_copy(k, b).wait()
        j = k + PF

        @pl.when(j < n_chunks)
        def _prefetch():
          row_copy(j, (b + PF) % NBUF).start()

        accum_chunk(b, k)
    return 0
  lax.fori_loop(0, (n_chunks + NBUF - 1) // NBUF, ring_body, 0)

  @pl.when(tid == NS - 1)
  def _tail():
    pltpu.sync_copy(btail_hbm.at[pl.ds(0, TAIL)],
                    ids_v.at[LPT, pl.ds(0, TAIL)])
    pltpu.sync_copy(x_hbm.at[pl.ds(TAIL_OFF, TAIL), pl.ds(c0, DH)],
                    rows_v.at[0, pl.ds(0, TAIL)])
    accum_groups(0, LPT, TAIL)

  for h in range(2):
    pltpu.sync_copy(acc_v.at[pl.ds(h * CH, CH)], acc_sh.at[idx_v.at[h]],
                    add=True)

  plsc.subcore_barrier()

  g0 = tid * GPT
  pltpu.sync_copy(acc_sh.at[pl.ds(g0, GPT)], acc_v.at[pl.ds(0, GPT)])

  def div_body(r, _):
    cnt = acc_v[r, pl.ds(DH, L)]
    for q in range(DH // L):
      cnt_v[r, pl.ds(q * L, L)] = acc_v[r, pl.ds(q * L, L)] / cnt
    return 0
  lax.fori_loop(0, GPT, div_body, 0)

  pltpu.sync_copy(cnt_v.at[pl.ds(0, GPT)],
                  out_hbm.at[pl.ds(g0, GPT), pl.ds(c0, DH)])


def kernel(x, batch):
  b2d = batch[:TAIL_OFF].reshape(NFULL, CH)
  btail = batch[TAIL_OFF:]
  mesh = plsc.VectorSubcoreMesh(core_axis_name="c", subcore_axis_name="s")
  return pl.kernel(
      _seg_mean_kernel,
      out_type=jax.ShapeDtypeStruct((G, D), jnp.float32),
      mesh=mesh,
      scratch_types=[
          pltpu.VMEM_SHARED((G, AW), jnp.float32),
          pltpu.VMEM((G, AW), jnp.float32),
          pltpu.VMEM((NBUF, CH, DH), jnp.float32),
          pltpu.VMEM((CPT, CH), jnp.int32),
          pltpu.VMEM((2, CH), jnp.int32),
          pltpu.VMEM((GPT, DH), jnp.float32),
          pltpu.SemaphoreType.DMA,
          pltpu.SemaphoreType.DMA,
          pltpu.SemaphoreType.DMA,
          pltpu.SemaphoreType.DMA,
          pltpu.SemaphoreType.DMA,
          pltpu.SemaphoreType.DMA,
          pltpu.SemaphoreType.DMA,
          pltpu.SemaphoreType.DMA,
      ],
      compiler_params=pltpu.CompilerParams(use_tc_tiling_on_sc=False,
                                           needs_layout_passes=False),
  )(x, b2d, btail)


# speedup vs baseline: 1.9664x; 1.9664x over previous
import jax
import jax.numpy as jnp
from jax import lax
from jax.experimental import pallas as pl
from jax.experimental.pallas import tpu as pltpu
from jax.experimental.pallas import tpu_sc as plsc

N = 100000
D = 128
G = 256

NC = 2
NS = 16
L = 16
DH = D // NC
CH = 128
NBUF = 6
PF = 3
UNROLL = 8

NFULL = N // CH
TAIL = N - NFULL * CH
TAIL_OFF = NFULL * CH
CPT = 49
LPT = NFULL - (NS - 1) * CPT
AW = DH + L
ZR = G // NS
GPT = G // NS


def _seg_mean_kernel(x_hbm, b2d_hbm, btail_hbm, out_hbm,
                     acc_sh, acc_v, rows_v, ids_v, idx_v, cnt_v,
                     sem_i, sem0, sem1, sem2, sem3, sem4, sem5, sem_s):
  tid = lax.axis_index("s")
  cid = lax.axis_index("c")
  c0 = cid * DH
  chunk0 = tid * CPT
  n_chunks = jnp.where(tid < NS - 1, CPT, LPT)
  sems = (sem0, sem1, sem2, sem3, sem4, sem5)
  zero16 = jnp.zeros((L,), jnp.float32)
  one16 = jnp.ones((L,), jnp.float32)

  @pl.when(tid < NS - 1)
  def _ids_full():
    pltpu.async_copy(b2d_hbm.at[pl.ds(chunk0, CPT)], ids_v.at[pl.ds(0, CPT)],
                     sem_i)

  @pl.when(tid == NS - 1)
  def _ids_last():
    pltpu.async_copy(b2d_hbm.at[pl.ds(chunk0, LPT)], ids_v.at[pl.ds(0, LPT)],
                     sem_i)

  def zero_body(r, _):
    for q in range(AW // L):
      acc_v[r, pl.ds(q * L, L)] = zero16
    return 0
  lax.fori_loop(0, G, zero_body, 0)
  pltpu.sync_copy(acc_v.at[pl.ds(0, ZR)], acc_sh.at[pl.ds(tid * ZR, ZR)])

  base = jnp.arange(L, dtype=jnp.int32)
  for h in range(2):
    for r in range(CH // L):
      idx_v[h, pl.ds(r * L, L)] = base + (h * CH + r * L)

  @pl.when(tid < NS - 1)
  def _ids_full_wait():
    pltpu.make_async_copy(b2d_hbm.at[pl.ds(chunk0, CPT)],
                          ids_v.at[pl.ds(0, CPT)], sem_i).wait()

  @pl.when(tid == NS - 1)
  def _ids_last_wait():
    pltpu.make_async_copy(b2d_hbm.at[pl.ds(chunk0, LPT)],
                          ids_v.at[pl.ds(0, LPT)], sem_i).wait()

  plsc.subcore_barrier()

  def row_copy(k, b):
    off = (chunk0 + k) * CH
    return pltpu.make_async_copy(x_hbm.at[pl.ds(off, CH), pl.ds(c0, DH)],
                                 rows_v.at[b], sems[b])

  for b in range(PF):
    row_copy(b, b).start()

  sixteen16 = jnp.full((L,), float(L), jnp.float32)
  chunk16 = jnp.full((L,), float(CH), jnp.float32)
  zq = (zero16,) * (DH // L)

  def accum_groups(b, kk, nrows):
    def acc_body(ri, _):
      idvec = ids_v[kk, pl.ds(ri * L, L)]
      sid0 = jnp.max(idvec)
      uniform = sid0 == jnp.min(idvec)

      @pl.when(uniform)
      def _fast():
        for q in range(DH // L):
          vals = [rows_v[b, ri * L + u, pl.ds(q * L, L)] for u in range(L)]
          while len(vals) > 1:
            vals = [vals[i] + vals[i + 1] for i in range(0, len(vals), 2)]
          plsc.addupdate(acc_v.at[sid0, pl.ds(q * L, L)], vals[0])
        plsc.addupdate(acc_v.at[sid0, pl.ds(DH, L)], sixteen16)

      @pl.when(jnp.logical_not(uniform))
      def _slow():
        for u in range(L):
          r = ri * L + u
          sid = idvec[u]
          for q in range(DH // L):
            plsc.addupdate(acc_v.at[sid, pl.ds(q * L, L)],
                           rows_v[b, r, pl.ds(q * L, L)])
          plsc.addupdate(acc_v.at[sid, pl.ds(DH, L)], one16)
      return 0
    lax.fori_loop(0, nrows // L, acc_body, 0)

  def accum_chunk(b, kk):
    first = jnp.min(ids_v[kk, pl.ds(0, L)])
    last = jnp.max(ids_v[kk, pl.ds(CH - L, L)])

    @pl.when(first == last)
    def _chunk_fast():
      def fsum(ri, _):
        for q in range(DH // L):
          vals = [rows_v[b, ri * 2 * L + u, pl.ds(q * L, L)]
                  for u in range(2 * L)]
          while len(vals) > 1:
            vals = [vals[i] + vals[i + 1] for i in range(0, len(vals), 2)]
          plsc.addupdate(acc_v.at[last, pl.ds(q * L, L)], vals[0])
        return 0
      lax.fori_loop(0, CH // (2 * L), fsum, 0)
      plsc.addupdate(acc_v.at[last, pl.ds(DH, L)], chunk16)

    @pl.when(first != last)
    def _chunk_slow():
      accum_groups(b, kk, CH)

  def ring_body(i, _):
    for b in range(NBUF):
      k = NBUF * i + b

      @pl.when(k < n_chunks)
      def _process():
        row_copy(k, b).wait()
        j = k + PF

        @pl.when(j < n_chunks)
        def _prefetch():
          row_copy(j, (b + PF) % NBUF).start()

        accum_chunk(b, k)
    return 0
  lax.fori_loop(0, (n_chunks + NBUF - 1) // NBUF, ring_body, 0)

  @pl.when(tid == NS - 1)
  def _tail():
    pltpu.sync_copy(btail_hbm.at[pl.ds(0, TAIL)],
                    ids_v.at[LPT, pl.ds(0, TAIL)])
    pltpu.sync_copy(x_hbm.at[pl.ds(TAIL_OFF, TAIL), pl.ds(c0, DH)],
                    rows_v.at[0, pl.ds(0, TAIL)])
    accum_groups(0, LPT, TAIL)

  for h in range(2):
    pltpu.sync_copy(acc_v.at[pl.ds(h * CH, CH)], acc_sh.at[idx_v.at[h]],
                    add=True)

  plsc.subcore_barrier()

  g0 = tid * GPT
  pltpu.sync_copy(acc_sh.at[pl.ds(g0, GPT)], acc_v.at[pl.ds(0, GPT)])

  def div_body(r, _):
    cnt = acc_v[r, pl.ds(DH, L)]
    for q in range(DH // L):
      cnt_v[r, pl.ds(q * L, L)] = acc_v[r, pl.ds(q * L, L)] / cnt
    return 0
  lax.fori_loop(0, GPT, div_body, 0)

  pltpu.sync_copy(cnt_v.at[pl.ds(0, GPT)],
                  out_hbm.at[pl.ds(g0, GPT), pl.ds(c0, DH)])


def kernel(x, batch):
  b2d = batch[:TAIL_OFF].reshape(NFULL, CH)
  btail = batch[TAIL_OFF:]
  mesh = plsc.VectorSubcoreMesh(core_axis_name="c", subcore_axis_name="s")
  return pl.kernel(
      _seg_mean_kernel,
      out_type=jax.ShapeDtypeStruct((G, D), jnp.float32),
      mesh=mesh,
      scratch_types=[
          pltpu.VMEM_SHARED((G, AW), jnp.float32),
          pltpu.VMEM((G, AW), jnp.float32),
          pltpu.VMEM((NBUF, CH, DH), jnp.float32),
          pltpu.VMEM((CPT, CH), jnp.int32),
          pltpu.VMEM((2, CH), jnp.int32),
          pltpu.VMEM((GPT, DH), jnp.float32),
          pltpu.SemaphoreType.DMA,
          pltpu.SemaphoreType.DMA,
          pltpu.SemaphoreType.DMA,
          pltpu.SemaphoreType.DMA,
          pltpu.SemaphoreType.DMA,
          pltpu.SemaphoreType.DMA,
          pltpu.SemaphoreType.DMA,
          pltpu.SemaphoreType.DMA,
      ],
      compiler_params=pltpu.CompilerParams(use_tc_tiling_on_sc=False,
                                           needs_layout_passes=False),
  )(x, b2d, btail)


# speedup vs baseline: 2.3173x; 1.1785x over previous
import jax
import jax.numpy as jnp
from jax import lax
from jax.experimental import pallas as pl
from jax.experimental.pallas import tpu as pltpu
from jax.experimental.pallas import tpu_sc as plsc

N = 100000
D = 128
G = 256

NC = 2
NS = 16
L = 16
DH = D // NC
CH = 128
NBUF = 6
PF = 3

NFULL = N // CH
TAIL = N - NFULL * CH
TAIL_OFF = NFULL * CH
CPT = 49
LPT = NFULL - (NS - 1) * CPT
AW = DH + L
ZR = G // NS
GPT = G // NS


def _seg_mean_kernel(x_hbm, b2d_hbm, btail_hbm, out_hbm,
                     acc_sh, acc_v, rows_v, ids_v, idx_v, cnt_v,
                     sem_i, sem0, sem1, sem2, sem3, sem4, sem5):
  tid = lax.axis_index("s")
  cid = lax.axis_index("c")
  c0 = cid * DH
  chunk0 = tid * CPT
  n_chunks = jnp.where(tid < NS - 1, CPT, LPT)
  sems = (sem0, sem1, sem2, sem3, sem4, sem5)
  zero16 = jnp.zeros((L,), jnp.float32)
  one16 = jnp.ones((L,), jnp.float32)

  @pl.when(tid < NS - 1)
  def _ids_full():
    pltpu.async_copy(b2d_hbm.at[pl.ds(chunk0, CPT)], ids_v.at[pl.ds(0, CPT)],
                     sem_i)

  @pl.when(tid == NS - 1)
  def _ids_last():
    pltpu.async_copy(b2d_hbm.at[pl.ds(chunk0, LPT)], ids_v.at[pl.ds(0, LPT)],
                     sem_i)

  def zero_body(r, _):
    for q in range(AW // L):
      acc_v[r, pl.ds(q * L, L)] = zero16
    return 0
  lax.fori_loop(0, G, zero_body, 0)
  pltpu.sync_copy(acc_v.at[pl.ds(0, ZR)], acc_sh.at[pl.ds(tid * ZR, ZR)])

  base = jnp.arange(L, dtype=jnp.int32)
  for h in range(2):
    for r in range(CH // L):
      idx_v[h, pl.ds(r * L, L)] = base + (h * CH + r * L)

  @pl.when(tid < NS - 1)
  def _ids_full_wait():
    pltpu.make_async_copy(b2d_hbm.at[pl.ds(chunk0, CPT)],
                          ids_v.at[pl.ds(0, CPT)], sem_i).wait()

  @pl.when(tid == NS - 1)
  def _ids_last_wait():
    pltpu.make_async_copy(b2d_hbm.at[pl.ds(chunk0, LPT)],
                          ids_v.at[pl.ds(0, LPT)], sem_i).wait()

  plsc.subcore_barrier()

  def row_copy(k, b):
    off = (chunk0 + k) * CH
    return pltpu.make_async_copy(x_hbm.at[pl.ds(off, CH), pl.ds(c0, DH)],
                                 rows_v.at[b], sems[b])

  for b in range(PF):
    row_copy(b, b).start()

  sixteen16 = jnp.full((L,), float(L), jnp.float32)
  chunk16 = jnp.full((L,), float(CH), jnp.float32)

  def accum_groups(b, kk, nrows):
    def acc_body(ri, _):
      idvec = ids_v[kk, pl.ds(ri * L, L)]
      sid0 = jnp.max(idvec)
      uniform = sid0 == jnp.min(idvec)

      @pl.when(uniform)
      def _fast():
        for q in range(DH // L):
          vals = [rows_v[b, ri * L + u, pl.ds(q * L, L)] for u in range(L)]
          while len(vals) > 1:
            vals = [vals[i] + vals[i + 1] for i in range(0, len(vals), 2)]
          plsc.addupdate(acc_v.at[sid0, pl.ds(q * L, L)], vals[0])
        plsc.addupdate(acc_v.at[sid0, pl.ds(DH, L)], sixteen16)

      @pl.when(jnp.logical_not(uniform))
      def _slow():
        for u in range(L):
          r = ri * L + u
          sid = idvec[u]
          for q in range(DH // L):
            plsc.addupdate(acc_v.at[sid, pl.ds(q * L, L)],
                           rows_v[b, r, pl.ds(q * L, L)])
          plsc.addupdate(acc_v.at[sid, pl.ds(DH, L)], one16)
      return 0
    lax.fori_loop(0, nrows // L, acc_body, 0)

  def accum_chunk(b, kk):
    first = jnp.min(ids_v[kk, pl.ds(0, L)])
    last = jnp.max(ids_v[kk, pl.ds(CH - L, L)])

    @pl.when(first == last)
    def _chunk_fast():
      def fsum(ri, _):
        for q in range(DH // L):
          vals = [rows_v[b, ri * L + u, pl.ds(q * L, L)] for u in range(L)]
          while len(vals) > 1:
            vals = [vals[i] + vals[i + 1] for i in range(0, len(vals), 2)]
          plsc.addupdate(acc_v.at[last, pl.ds(q * L, L)], vals[0])
        return 0
      lax.fori_loop(0, CH // L, fsum, 0)
      plsc.addupdate(acc_v.at[last, pl.ds(DH, L)], chunk16)

    @pl.when(first != last)
    def _chunk_slow():
      accum_groups(b, kk, CH)

  def ring_body(i, _):
    for b in range(NBUF):
      k = NBUF * i + b

      @pl.when(k < n_chunks)
      def _process():
        row_copy(k, b).wait()
        j = k + PF

        @pl.when(j < n_chunks)
        def _prefetch():
          row_copy(j, (b + PF) % NBUF).start()

        accum_chunk(b, k)
    return 0
  lax.fori_loop(0, (n_chunks + NBUF - 1) // NBUF, ring_body, 0)

  @pl.when(tid == NS - 1)
  def _tail():
    pltpu.sync_copy(btail_hbm.at[pl.ds(0, TAIL)],
                    ids_v.at[LPT, pl.ds(0, TAIL)])
    pltpu.sync_copy(x_hbm.at[pl.ds(TAIL_OFF, TAIL), pl.ds(c0, DH)],
                    rows_v.at[0, pl.ds(0, TAIL)])
    accum_groups(0, LPT, TAIL)

  for h in range(2):
    pltpu.sync_copy(acc_v.at[pl.ds(h * CH, CH)], acc_sh.at[idx_v.at[h]],
                    add=True)

  plsc.subcore_barrier()

  g0 = tid * GPT
  pltpu.sync_copy(acc_sh.at[pl.ds(g0, GPT)], acc_v.at[pl.ds(0, GPT)])

  def div_body(r, _):
    cnt = acc_v[r, pl.ds(DH, L)]
    for q in range(DH // L):
      cnt_v[r, pl.ds(q * L, L)] = acc_v[r, pl.ds(q * L, L)] / cnt
    return 0
  lax.fori_loop(0, GPT, div_body, 0)

  pltpu.sync_copy(cnt_v.at[pl.ds(0, GPT)],
                  out_hbm.at[pl.ds(g0, GPT), pl.ds(c0, DH)])


def kernel(x, batch):
  b2d = batch[:TAIL_OFF].reshape(NFULL, CH)
  btail = batch[TAIL_OFF:]
  mesh = plsc.VectorSubcoreMesh(core_axis_name="c", subcore_axis_name="s")
  return pl.kernel(
      _seg_mean_kernel,
      out_type=jax.ShapeDtypeStruct((G, D), jnp.float32),
      mesh=mesh,
      scratch_types=[
          pltpu.VMEM_SHARED((G, AW), jnp.float32),
          pltpu.VMEM((G, AW), jnp.float32),
          pltpu.VMEM((NBUF, CH, DH), jnp.float32),
          pltpu.VMEM((CPT, CH), jnp.int32),
          pltpu.VMEM((2, CH), jnp.int32),
          pltpu.VMEM((GPT, DH), jnp.float32),
          pltpu.SemaphoreType.DMA,
          pltpu.SemaphoreType.DMA,
          pltpu.SemaphoreType.DMA,
          pltpu.SemaphoreType.DMA,
          pltpu.SemaphoreType.DMA,
          pltpu.SemaphoreType.DMA,
          pltpu.SemaphoreType.DMA,
      ],
      compiler_params=pltpu.CompilerParams(use_tc_tiling_on_sc=False,
                                           needs_layout_passes=False),
  )(x, b2d, btail)
